# baseline (speedup 1.0000x reference)
import functools
import jax
import jax.numpy as jnp
from jax import lax
from jax.experimental import pallas as pl
from jax.experimental.pallas import tpu as pltpu, tpu_sc as plsc

_R = 1024
_C = 26
_SIZE = 1000
_NW = 32
_TR = _SIZE // 8
_NSLAB = _C * _TR
_SLAB = 8192


def _slab_compute(idx_v, buf, c, tr):
    k0 = tr * 8

    def _tc_body(tc, _):
        base = c * _R + tc * 128
        ivs = [idx_v[pl.ds(base + g * 16, 16)] for g in range(8)]
        for ks in range(8):
            kvec = jnp.full((16,), k0 + ks, jnp.int32)
            for g in range(8):
                buf[pl.ds(tc * 1024 + ks * 128 + g * 16, 16)] = (
                    ivs[g] == kvec
                ).astype(jnp.float32)
        return _

    lax.fori_loop(0, 8, _tc_body, 0)


def _advance(c, tr, step):
    tr = tr + step
    wrap = (tr >= _TR).astype(jnp.int32)
    return c + wrap, tr - wrap * _TR


def _sc_body(idx_hbm, out_hbm, idx_v, buf_a, buf_b, sem_a, sem_b):
    nc = 2
    w = lax.axis_index("s") * nc + lax.axis_index("c")

    pltpu.sync_copy(idx_hbm, idx_v)

    def _start(buf, sem, s):
        return pltpu.async_copy(
            buf, out_hbm.at[pl.ds(s * _SLAB, _SLAB)], sem
        )

    def _drain(buf, sem):
        pltpu.make_async_copy(
            buf, out_hbm.at[pl.ds(0, _SLAB)], sem
        ).wait()

    c0 = jnp.int32(0)
    tr0 = w.astype(jnp.int32)

    _slab_compute(idx_v, buf_a, c0, tr0)
    _start(buf_a, sem_a, w)
    c1, tr1 = _advance(c0, tr0, 32)
    _slab_compute(idx_v, buf_b, c1, tr1)
    _start(buf_b, sem_b, w + 32)

    def _loop_body(j, carry):
        c, tr, s = carry
        c, tr = _advance(c, tr, 32)
        s = s + 32
        _drain(buf_a, sem_a)
        _slab_compute(idx_v, buf_a, c, tr)
        _start(buf_a, sem_a, s)
        c, tr = _advance(c, tr, 32)
        s = s + 32
        _drain(buf_b, sem_b)
        _slab_compute(idx_v, buf_b, c, tr)
        _start(buf_b, sem_b, s)
        return c, tr, s

    c2, tr2, s2 = lax.fori_loop(
        1, 50, _loop_body, (c1, tr1, w + 32)
    )

    c3, tr3 = _advance(c2, tr2, 32)
    _drain(buf_a, sem_a)
    _slab_compute(idx_v, buf_a, c3, tr3)
    _start(buf_a, sem_a, s2 + 32)

    @pl.when(w < _NSLAB - 3232)
    def _extra():
        _drain(buf_b, sem_b)
        _slab_compute(idx_v, buf_b, jnp.int32(25), (107 + w).astype(jnp.int32))
        _start(buf_b, sem_b, 3232 + w)

    _drain(buf_a, sem_a)
    _drain(buf_b, sem_b)


_sc_onehot = functools.partial(
    pl.kernel,
    mesh=plsc.VectorSubcoreMesh(core_axis_name="c", subcore_axis_name="s"),
    out_type=jax.ShapeDtypeStruct((_R * _C * _SIZE,), jnp.float32),
    compiler_params=pltpu.CompilerParams(needs_layout_passes=False),
    scratch_types=[
        pltpu.VMEM((_R * _C,), jnp.int32),
        pltpu.VMEM((_SLAB,), jnp.float32),
        pltpu.VMEM((_SLAB,), jnp.float32),
        pltpu.SemaphoreType.DMA,
        pltpu.SemaphoreType.DMA,
    ],
)(_sc_body)


def kernel(x, size):
    del size
    idx_t = x.astype(jnp.int32).T.reshape(_C * _R)
    out = _sc_onehot(idx_t)
    return (
        out.reshape(_C, _TR, 8, 8, 128)
        .transpose(2, 4, 0, 1, 3)
        .reshape(_R, _C, _SIZE)
    )


# speedup vs baseline: 1.0625x; 1.0625x over previous
import functools
import jax
import jax.numpy as jnp
from jax import lax
from jax.experimental import pallas as pl
from jax.experimental.pallas import tpu as pltpu, tpu_sc as plsc

_R = 1024
_C = 26
_SIZE = 1000
_TR = _SIZE // 8
_NSLAB = _C * _TR
_SLAB = 8192
_NBUF = 4


def _divmod125(s):
    c = (s * 8389) >> 20
    return c, s - c * _TR


def _slab_compute(idx_v, buf, c_local, tr):
    k0 = tr * 8

    def _tc_body(tc, _):
        base = c_local * _R + tc * 128
        ivs = [idx_v[pl.ds(base + g * 16, 16)] for g in range(8)]
        for ks in range(8):
            kvec = jnp.full((16,), k0 + ks, jnp.int32)
            for g in range(8):
                buf[pl.ds(tc * 1024 + ks * 128 + g * 16, 16)] = (
                    ivs[g] == kvec
                ).astype(jnp.float32)
        return _

    lax.fori_loop(0, 8, _tc_body, 0)


def _advance(c, tr):
    wrap = (tr + 1 >= _TR).astype(jnp.int32)
    return c + wrap, (tr + 1) - wrap * _TR


def _sc_body(idx_hbm, out_hbm, idx_v, *bufs_and_sems):
    bufs = bufs_and_sems[:_NBUF]
    sems = bufs_and_sems[_NBUF:]
    nc = 2
    w = lax.axis_index("s") * nc + lax.axis_index("c")

    base = 101 * w + jnp.minimum(w, 18)
    c0, tr0 = _divmod125(base)

    pltpu.sync_copy(idx_hbm.at[pl.ds(c0 * _R, 2 * _R)], idx_v)

    def _start(buf, sem, s):
        return pltpu.async_copy(buf, out_hbm.at[pl.ds(s * _SLAB, _SLAB)], sem)

    def _drain(buf, sem):
        pltpu.make_async_copy(buf, out_hbm.at[pl.ds(0, _SLAB)], sem).wait()

    c, tr = c0, tr0
    for b in range(_NBUF):
        _slab_compute(idx_v, bufs[b], c - c0, tr)
        _start(bufs[b], sems[b], base + b)
        c, tr = _advance(c, tr)

    def _loop_body(j, carry):
        c, tr = carry
        for b in range(_NBUF):
            s = base + _NBUF * j + b
            _drain(bufs[b], sems[b])
            _slab_compute(idx_v, bufs[b], c - c0, tr)
            _start(bufs[b], sems[b], s)
            c, tr = _advance(c, tr)
        return c, tr

    c, tr = lax.fori_loop(1, 25, _loop_body, (c, tr))

    _drain(bufs[0], sems[0])
    _slab_compute(idx_v, bufs[0], c - c0, tr)
    _start(bufs[0], sems[0], base + 100)
    c, tr = _advance(c, tr)

    @pl.when(w < 18)
    def _extra():
        _drain(bufs[1], sems[1])
        _slab_compute(idx_v, bufs[1], c - c0, tr)
        _start(bufs[1], sems[1], base + 101)

    for b in range(_NBUF):
        _drain(bufs[b], sems[b])


_sc_onehot = functools.partial(
    pl.kernel,
    mesh=plsc.VectorSubcoreMesh(core_axis_name="c", subcore_axis_name="s"),
    out_type=jax.ShapeDtypeStruct((_R * _C * _SIZE,), jnp.float32),
    compiler_params=pltpu.CompilerParams(needs_layout_passes=False),
    scratch_types=[
        pltpu.VMEM((2 * _R,), jnp.int32),
        *([pltpu.VMEM((_SLAB,), jnp.float32)] * _NBUF),
        *([pltpu.SemaphoreType.DMA] * _NBUF),
    ],
)(_sc_body)


def kernel(x, size):
    del size
    idx_t = x.astype(jnp.int32).T.reshape(_C * _R)
    idx_t = jnp.pad(idx_t, (0, _R))
    out = _sc_onehot(idx_t)
    return (
        out.reshape(_C, _TR, 8, 8, 128)
        .transpose(2, 4, 0, 1, 3)
        .reshape(_R, _C, _SIZE)
    )
